# Initial kernel scaffold; baseline (speedup 1.0000x reference)
import functools

import jax
import jax.numpy as jnp
from jax import lax
from jax.experimental import pallas as pl
from jax.experimental.pallas import tpu as pltpu
from jax.experimental.pallas import tpu_sc as plsc

_N = 10000
_HID = 256

_KB = 128
_NBLK = 79
_EW = _NBLK * _KB
_NW = 32
_NPAD = 10240
_SLAB = _NPAD // 16
_DUMP = 10000

_BN = 1000
_GRID = _N // _BN

_PREC = lax.Precision.HIGHEST


def _mm(x, w, b=None):
    y = lax.dot_general(x, w, (((1,), (0,)), ((), ())),
                        precision=_PREC, preferred_element_type=jnp.float32)
    if b is not None:
        y = y + b
    return y


def _sc_agg_body(v1a, v1b, srcs, dsts, out, src_v, dst_v, stage_v, zero_v,
                 zstage_v, acc_sh, sem):
    c = lax.axis_index("c")
    s = lax.axis_index("s")
    wid = s * 2 + c
    pltpu.sync_copy(srcs.at[wid], src_v)
    pltpu.sync_copy(dsts.at[wid], dst_v)

    zv = jnp.zeros((16,), jnp.float32)
    for r in range(16):
        for q in range(8):
            zero_v[r, pl.ds(q * 16, 16)] = zv
    for r in range(8):
        pltpu.sync_copy(zero_v, zstage_v.at[pl.ds(r * 16, 16)])

    for chunk in range(2):
        table = v1a if chunk == 0 else v1b

        for r in range(_SLAB // _KB):
            pltpu.sync_copy(zstage_v, acc_sh.at[pl.ds(s * _SLAB + r * _KB, _KB)])
        plsc.subcore_barrier()

        def ebody(j, carry):
            pltpu.async_copy(table.at[src_v.at[j]], stage_v, sem).wait()
            pltpu.sync_copy(stage_v, acc_sh.at[dst_v.at[j]], add=True)
            return carry

        lax.fori_loop(0, _NBLK, ebody, 0)
        plsc.subcore_barrier()

        pltpu.sync_copy(acc_sh.at[pl.ds(s * _SLAB, _SLAB)],
                        out.at[c, chunk, pl.ds(s * _SLAB, _SLAB)])
        plsc.subcore_barrier()


def _sc_agg(v1a, v1b, srcs, dsts):
    mesh = plsc.VectorSubcoreMesh(core_axis_name="c", subcore_axis_name="s")
    fn = pl.kernel(
        _sc_agg_body,
        mesh=mesh,
        out_type=jax.ShapeDtypeStruct((2, 2, _NPAD, 128), jnp.float32),
        scratch_types=[
            pltpu.VMEM((_NBLK, _KB), jnp.int32),
            pltpu.VMEM((_NBLK, _KB), jnp.int32),
            pltpu.VMEM((_KB, 128), jnp.float32),
            pltpu.VMEM((16, 128), jnp.float32),
            pltpu.VMEM((_KB, 128), jnp.float32),
            pltpu.VMEM_SHARED((_NPAD, 128), jnp.float32),
            pltpu.SemaphoreType.DMA,
        ],
    )
    return fn(v1a, v1b, srcs, dsts)


def _pn1_body(v_ref, wp_ref, bp_ref, w0_ref, b0_ref, w1_ref, b1_ref, ws_ref,
              out_ref, max_ref):
    i = pl.program_id(0)
    x = _mm(v_ref[...], wp_ref[...], bp_ref[...])
    net = _mm(jnp.maximum(x, 0.0), w0_ref[...], b0_ref[...])
    dx = _mm(jnp.maximum(net, 0.0), w1_ref[...], b1_ref[...])
    out = _mm(x, ws_ref[...]) + dx
    out_ref[...] = out
    m = jnp.max(out, axis=0, keepdims=True)

    @pl.when(i == 0)
    def _():
        max_ref[...] = m

    @pl.when(i > 0)
    def _():
        max_ref[...] = jnp.maximum(max_ref[...], m)


def _pn1(verts, wp, bp, w0, b0, w1, b1, ws):
    return pl.pallas_call(
        _pn1_body,
        grid=(_GRID,),
        in_specs=[
            pl.BlockSpec((_BN, 3), lambda i: (i, 0)),
            pl.BlockSpec((3, 512), lambda i: (0, 0)),
            pl.BlockSpec((1, 512), lambda i: (0, 0)),
            pl.BlockSpec((512, 256), lambda i: (0, 0)),
            pl.BlockSpec((1, 256), lambda i: (0, 0)),
            pl.BlockSpec((256, 256), lambda i: (0, 0)),
            pl.BlockSpec((1, 256), lambda i: (0, 0)),
            pl.BlockSpec((512, 256), lambda i: (0, 0)),
        ],
        out_specs=[
            pl.BlockSpec((_BN, 256), lambda i: (i, 0)),
            pl.BlockSpec((1, 256), lambda i: (0, 0)),
        ],
        out_shape=[
            jax.ShapeDtypeStruct((_N, 256), jnp.float32),
            jax.ShapeDtypeStruct((1, 256), jnp.float32),
        ],
    )(verts, wp, bp, w0, b0, w1, b1, ws)


def _pnk_body(net_ref, pool_ref, w0_ref, b0_ref, w1_ref, b1_ref, ws_ref,
              out_ref, max_ref):
    i = pl.program_id(0)
    a = net_ref[...]
    x = jnp.concatenate([a, jnp.broadcast_to(pool_ref[...], a.shape)], axis=1)
    net = _mm(jnp.maximum(x, 0.0), w0_ref[...], b0_ref[...])
    dx = _mm(jnp.maximum(net, 0.0), w1_ref[...], b1_ref[...])
    out = _mm(x, ws_ref[...]) + dx
    out_ref[...] = out
    m = jnp.max(out, axis=0, keepdims=True)

    @pl.when(i == 0)
    def _():
        max_ref[...] = m

    @pl.when(i > 0)
    def _():
        max_ref[...] = jnp.maximum(max_ref[...], m)


def _pnk(net, pooled, w0, b0, w1, b1, ws):
    return pl.pallas_call(
        _pnk_body,
        grid=(_GRID,),
        in_specs=[
            pl.BlockSpec((_BN, 256), lambda i: (i, 0)),
            pl.BlockSpec((1, 256), lambda i: (0, 0)),
            pl.BlockSpec((512, 256), lambda i: (0, 0)),
            pl.BlockSpec((1, 256), lambda i: (0, 0)),
            pl.BlockSpec((256, 256), lambda i: (0, 0)),
            pl.BlockSpec((1, 256), lambda i: (0, 0)),
            pl.BlockSpec((512, 256), lambda i: (0, 0)),
        ],
        out_specs=[
            pl.BlockSpec((_BN, 256), lambda i: (i, 0)),
            pl.BlockSpec((1, 256), lambda i: (0, 0)),
        ],
        out_shape=[
            jax.ShapeDtypeStruct((_N, 256), jnp.float32),
            jax.ShapeDtypeStruct((1, 256), jnp.float32),
        ],
    )(net, pooled, w0, b0, w1, b1, ws)


def _enc_body(mx_ref, wc_ref, bc_ref, w0r_ref, b0_ref, w1r_ref, b1_ref,
              b0e_ref, b1e_ref):
    enc = _mm(jnp.maximum(mx_ref[...], 0.0), wc_ref[...], bc_ref[...])
    b0e_ref[...] = b0_ref[...] + _mm(enc, w0r_ref[...])
    b1e_ref[...] = b1_ref[...] + _mm(enc, w1r_ref[...])


def _enc(mx, wc, bc, w0r, b0, w1r, b1):
    return pl.pallas_call(
        _enc_body,
        grid=(1,),
        in_specs=[
            pl.BlockSpec((1, 256), lambda i: (0, 0)),
            pl.BlockSpec((256, 256), lambda i: (0, 0)),
            pl.BlockSpec((1, 256), lambda i: (0, 0)),
            pl.BlockSpec((256, 256), lambda i: (0, 0)),
            pl.BlockSpec((1, 256), lambda i: (0, 0)),
            pl.BlockSpec((256, 256), lambda i: (0, 0)),
            pl.BlockSpec((1, 256), lambda i: (0, 0)),
        ],
        out_specs=[
            pl.BlockSpec((1, 256), lambda i: (0, 0)),
            pl.BlockSpec((1, 256), lambda i: (0, 0)),
        ],
        out_shape=[
            jax.ShapeDtypeStruct((1, 256), jnp.float32),
            jax.ShapeDtypeStruct((1, 256), jnp.float32),
        ],
    )(mx, wc, bc, w0r, b0, w1r, b1)


def _g0_body(v_ref, w0_ref, b0_ref, w1_ref, b1_ref, v0_ref, v1a_ref, v1b_ref):
    v = v_ref[...]
    v0_ref[...] = _mm(v, w0_ref[...], b0_ref[...])
    v1 = _mm(v, w1_ref[...], b1_ref[...])
    v1a_ref[...] = v1[:, :128]
    v1b_ref[...] = v1[:, 128:]


def _g0(verts, w0t, b0e, w1t, b1e):
    return pl.pallas_call(
        _g0_body,
        grid=(_GRID,),
        in_specs=[
            pl.BlockSpec((_BN, 3), lambda i: (i, 0)),
            pl.BlockSpec((3, 256), lambda i: (0, 0)),
            pl.BlockSpec((1, 256), lambda i: (0, 0)),
            pl.BlockSpec((3, 256), lambda i: (0, 0)),
            pl.BlockSpec((1, 256), lambda i: (0, 0)),
        ],
        out_specs=[
            pl.BlockSpec((_BN, 256), lambda i: (i, 0)),
            pl.BlockSpec((_BN, 128), lambda i: (i, 0)),
            pl.BlockSpec((_BN, 128), lambda i: (i, 0)),
        ],
        out_shape=[
            jax.ShapeDtypeStruct((_N, 256), jnp.float32),
            jax.ShapeDtypeStruct((_N, 128), jnp.float32),
            jax.ShapeDtypeStruct((_N, 128), jnp.float32),
        ],
    )(verts, w0t, b0e, w1t, b1e)


def _gt_body(v0p_ref, p_ref, w0_ref, b0_ref, w1_ref, b1_ref,
             v0_ref, v1a_ref, v1b_ref):
    p = p_ref[...]
    agg = jnp.concatenate([p[0, 0] + p[1, 0], p[0, 1] + p[1, 1]], axis=1)
    feat = jnp.maximum(v0p_ref[...] + agg, 0.0)
    v0_ref[...] = _mm(feat, w0_ref[...], b0_ref[...])
    v1 = _mm(feat, w1_ref[...], b1_ref[...])
    v1a_ref[...] = v1[:, :128]
    v1b_ref[...] = v1[:, 128:]


def _gt(v0p, parts, w0, b0, w1, b1):
    return pl.pallas_call(
        _gt_body,
        grid=(_GRID,),
        in_specs=[
            pl.BlockSpec((_BN, 256), lambda i: (i, 0)),
            pl.BlockSpec((2, 2, _BN, 128), lambda i: (0, 0, i, 0)),
            pl.BlockSpec((256, 256), lambda i: (0, 0)),
            pl.BlockSpec((1, 256), lambda i: (0, 0)),
            pl.BlockSpec((256, 256), lambda i: (0, 0)),
            pl.BlockSpec((1, 256), lambda i: (0, 0)),
        ],
        out_specs=[
            pl.BlockSpec((_BN, 256), lambda i: (i, 0)),
            pl.BlockSpec((_BN, 128), lambda i: (i, 0)),
            pl.BlockSpec((_BN, 128), lambda i: (i, 0)),
        ],
        out_shape=[
            jax.ShapeDtypeStruct((_N, 256), jnp.float32),
            jax.ShapeDtypeStruct((_N, 128), jnp.float32),
            jax.ShapeDtypeStruct((_N, 128), jnp.float32),
        ],
    )(v0p, parts, w0, b0, w1, b1)


def _head_body(v0p_ref, p_ref, w1_ref, b1_ref, w2_ref, b2_ref, w3_ref, b3_ref,
               w4_ref, b4_ref, w5_ref, b5_ref, out_ref):
    p = p_ref[...]
    agg = jnp.concatenate([p[0, 0] + p[1, 0], p[0, 1] + p[1, 1]], axis=1)
    x = jnp.maximum(v0p_ref[...] + agg, 0.0)
    x = jnp.maximum(_mm(x, w1_ref[...], b1_ref[...]), 0.0)
    x = jnp.maximum(_mm(x, w2_ref[...], b2_ref[...]), 0.0)
    x = jnp.maximum(_mm(x, w3_ref[...], b3_ref[...]), 0.0)
    x = jnp.maximum(_mm(x, w4_ref[...], b4_ref[...]), 0.0)
    out_ref[...] = _mm(x, w5_ref[...], b5_ref[...])


def _head(v0p, parts, ws):
    (w1, b1), (w2, b2), (w3, b3), (w4, b4), (w5, b5) = ws
    return pl.pallas_call(
        _head_body,
        grid=(_GRID,),
        in_specs=[
            pl.BlockSpec((_BN, 256), lambda i: (i, 0)),
            pl.BlockSpec((2, 2, _BN, 128), lambda i: (0, 0, i, 0)),
            pl.BlockSpec((256, 512), lambda i: (0, 0)),
            pl.BlockSpec((1, 512), lambda i: (0, 0)),
            pl.BlockSpec((512, 512), lambda i: (0, 0)),
            pl.BlockSpec((1, 512), lambda i: (0, 0)),
            pl.BlockSpec((512, 512), lambda i: (0, 0)),
            pl.BlockSpec((1, 512), lambda i: (0, 0)),
            pl.BlockSpec((512, 128), lambda i: (0, 0)),
            pl.BlockSpec((1, 128), lambda i: (0, 0)),
            pl.BlockSpec((128, 3), lambda i: (0, 0)),
            pl.BlockSpec((1, 3), lambda i: (0, 0)),
        ],
        out_specs=pl.BlockSpec((_BN, 3), lambda i: (i, 0)),
        out_shape=jax.ShapeDtypeStruct((_N, 3), jnp.float32),
    )(v0p, parts, w1, b1, w2, b2, w3, b3, w4, b4, w5, b5)


def _edge_plan(edges):
    i0 = edges[:, 0]
    i1 = edges[:, 1]
    dsts = jnp.concatenate([i0, i1])
    srcs = jnp.concatenate([i1, i0])
    pad = _NW * _EW - dsts.shape[0]
    dsts = jnp.concatenate([dsts, jnp.full((pad,), _DUMP, jnp.int32)])
    srcs = jnp.concatenate([srcs, jnp.zeros((pad,), jnp.int32)])
    return (srcs.reshape(_NW, _NBLK, _KB), dsts.reshape(_NW, _NBLK, _KB))


def _row(b):
    return b.reshape(1, -1)


def kernel(mesh_verts, edges, params):
    verts = mesh_verts.reshape(_N, 3)
    srcs, dsts = _edge_plan(edges)

    pn = params["pointnet"]
    blk = pn["blocks"]
    net, mx = _pn1(
        verts,
        pn["fc_pos"]["w"], _row(pn["fc_pos"]["b"]),
        blk[0]["fc_0"]["w"], _row(blk[0]["fc_0"]["b"]),
        blk[0]["fc_1"]["w"], _row(blk[0]["fc_1"]["b"]),
        blk[0]["shortcut"]["w"],
    )
    for k in range(1, 5):
        net, mx = _pnk(
            net, mx,
            blk[k]["fc_0"]["w"], _row(blk[k]["fc_0"]["b"]),
            blk[k]["fc_1"]["w"], _row(blk[k]["fc_1"]["b"]),
            blk[k]["shortcut"]["w"],
        )

    g = params["gconvs"]
    b0e, b1e = _enc(
        mx, pn["fc_c"]["w"], _row(pn["fc_c"]["b"]),
        g[0]["w0"]["w"][3:], _row(g[0]["w0"]["b"]),
        g[0]["w1"]["w"][3:], _row(g[0]["w1"]["b"]),
    )
    v0, v1a, v1b = _g0(verts, g[0]["w0"]["w"][:3], b0e, g[0]["w1"]["w"][:3], b1e)

    for t in range(1, 11):
        parts = _sc_agg(v1a, v1b, srcs, dsts)
        v0, v1a, v1b = _gt(
            v0, parts,
            g[t]["w0"]["w"], _row(g[t]["w0"]["b"]),
            g[t]["w1"]["w"], _row(g[t]["w1"]["b"]),
        )

    parts = _sc_agg(v1a, v1b, srcs, dsts)
    ws = [(p["w"], _row(p["b"])) for p in params["vert_offset"]]
    return _head(v0, parts, ws)


# speedup vs baseline: 1.9463x; 1.9463x over previous
import functools

import jax
import jax.numpy as jnp
from jax import lax
from jax.experimental import pallas as pl
from jax.experimental.pallas import tpu as pltpu
from jax.experimental.pallas import tpu_sc as plsc

_N = 10000
_HID = 256

_KB = 128
_NBLK = 79
_EW = _NBLK * _KB
_NW = 32
_NPAD = 10240
_SLAB = _NPAD // 16
_DUMP = 10000
_CW = 64

_BN = 1000
_GRID = _N // _BN

_PREC = None


def _mm(x, w, b=None):
    y = lax.dot_general(x, w, (((1,), (0,)), ((), ())),
                        precision=_PREC, preferred_element_type=jnp.float32)
    if b is not None:
        y = y + b
    return y


def _sc_agg_body(v1a, v1b, v1c, v1d, srcs, dsts, out, src_v, dst_v, stage_v,
                 zstage_v, acc_sh, sem):
    c = lax.axis_index("c")
    s = lax.axis_index("s")
    wid = s * 2 + c
    pltpu.sync_copy(srcs.at[wid], src_v)
    pltpu.sync_copy(dsts.at[wid], dst_v)

    zv = jnp.zeros((16,), jnp.float32)

    def zfill(r, carry):
        for q in range(_CW // 16):
            zstage_v[r, pl.ds(q * 16, 16)] = zv
        return carry

    lax.fori_loop(0, _KB, zfill, 0)

    for chunk, table in enumerate((v1a, v1b, v1c, v1d)):
        for r in range(_SLAB // _KB):
            pltpu.sync_copy(zstage_v, acc_sh.at[pl.ds(s * _SLAB + r * _KB, _KB)])
        plsc.subcore_barrier()

        def ebody(j, carry):
            pltpu.async_copy(table.at[src_v.at[j]], stage_v, sem).wait()
            pltpu.sync_copy(stage_v, acc_sh.at[dst_v.at[j]], add=True)
            return carry

        lax.fori_loop(0, _NBLK, ebody, 0)
        plsc.subcore_barrier()

        pltpu.sync_copy(acc_sh.at[pl.ds(s * _SLAB, _SLAB)],
                        out.at[c, chunk, pl.ds(s * _SLAB, _SLAB)])
        plsc.subcore_barrier()


def _sc_agg(v1a, v1b, v1c, v1d, srcs, dsts):
    mesh = plsc.VectorSubcoreMesh(core_axis_name="c", subcore_axis_name="s")
    fn = pl.kernel(
        _sc_agg_body,
        mesh=mesh,
        compiler_params=pltpu.CompilerParams(use_tc_tiling_on_sc=False),
        out_type=jax.ShapeDtypeStruct((2, 4, _NPAD, _CW), jnp.float32),
        scratch_types=[
            pltpu.VMEM((_NBLK, _KB), jnp.int32),
            pltpu.VMEM((_NBLK, _KB), jnp.int32),
            pltpu.VMEM((_KB, _CW), jnp.float32),
            pltpu.VMEM((_KB, _CW), jnp.float32),
            pltpu.VMEM_SHARED((_NPAD, _CW), jnp.float32),
            pltpu.SemaphoreType.DMA,
        ],
    )
    return fn(v1a, v1b, v1c, v1d, srcs, dsts)


def _pn1_body(v_ref, wp_ref, bp_ref, w0_ref, b0_ref, w1_ref, b1_ref, ws_ref,
              out_ref, max_ref):
    i = pl.program_id(0)
    x = _mm(v_ref[...], wp_ref[...], bp_ref[...])
    net = _mm(jnp.maximum(x, 0.0), w0_ref[...], b0_ref[...])
    dx = _mm(jnp.maximum(net, 0.0), w1_ref[...], b1_ref[...])
    out = _mm(x, ws_ref[...]) + dx
    out_ref[...] = out
    m = jnp.max(out, axis=0, keepdims=True)

    @pl.when(i == 0)
    def _():
        max_ref[...] = m

    @pl.when(i > 0)
    def _():
        max_ref[...] = jnp.maximum(max_ref[...], m)


def _pn1(verts, wp, bp, w0, b0, w1, b1, ws):
    return pl.pallas_call(
        _pn1_body,
        grid=(_GRID,),
        in_specs=[
            pl.BlockSpec((_BN, 3), lambda i: (i, 0)),
            pl.BlockSpec((3, 512), lambda i: (0, 0)),
            pl.BlockSpec((1, 512), lambda i: (0, 0)),
            pl.BlockSpec((512, 256), lambda i: (0, 0)),
            pl.BlockSpec((1, 256), lambda i: (0, 0)),
            pl.BlockSpec((256, 256), lambda i: (0, 0)),
            pl.BlockSpec((1, 256), lambda i: (0, 0)),
            pl.BlockSpec((512, 256), lambda i: (0, 0)),
        ],
        out_specs=[
            pl.BlockSpec((_BN, 256), lambda i: (i, 0)),
            pl.BlockSpec((1, 256), lambda i: (0, 0)),
        ],
        out_shape=[
            jax.ShapeDtypeStruct((_N, 256), jnp.float32),
            jax.ShapeDtypeStruct((1, 256), jnp.float32),
        ],
    )(verts, wp, bp, w0, b0, w1, b1, ws)


def _pnk_body(net_ref, pool_ref, w0_ref, b0_ref, w1_ref, b1_ref, ws_ref,
              out_ref, max_ref):
    i = pl.program_id(0)
    a = net_ref[...]
    x = jnp.concatenate([a, jnp.broadcast_to(pool_ref[...], a.shape)], axis=1)
    net = _mm(jnp.maximum(x, 0.0), w0_ref[...], b0_ref[...])
    dx = _mm(jnp.maximum(net, 0.0), w1_ref[...], b1_ref[...])
    out = _mm(x, ws_ref[...]) + dx
    out_ref[...] = out
    m = jnp.max(out, axis=0, keepdims=True)

    @pl.when(i == 0)
    def _():
        max_ref[...] = m

    @pl.when(i > 0)
    def _():
        max_ref[...] = jnp.maximum(max_ref[...], m)


def _pnk(net, pooled, w0, b0, w1, b1, ws):
    return pl.pallas_call(
        _pnk_body,
        grid=(_GRID,),
        in_specs=[
            pl.BlockSpec((_BN, 256), lambda i: (i, 0)),
            pl.BlockSpec((1, 256), lambda i: (0, 0)),
            pl.BlockSpec((512, 256), lambda i: (0, 0)),
            pl.BlockSpec((1, 256), lambda i: (0, 0)),
            pl.BlockSpec((256, 256), lambda i: (0, 0)),
            pl.BlockSpec((1, 256), lambda i: (0, 0)),
            pl.BlockSpec((512, 256), lambda i: (0, 0)),
        ],
        out_specs=[
            pl.BlockSpec((_BN, 256), lambda i: (i, 0)),
            pl.BlockSpec((1, 256), lambda i: (0, 0)),
        ],
        out_shape=[
            jax.ShapeDtypeStruct((_N, 256), jnp.float32),
            jax.ShapeDtypeStruct((1, 256), jnp.float32),
        ],
    )(net, pooled, w0, b0, w1, b1, ws)


def _enc_body(mx_ref, wc_ref, bc_ref, enc_ref):
    enc_ref[...] = _mm(jnp.maximum(mx_ref[...], 0.0), wc_ref[...], bc_ref[...])


def _enc(mx, wc, bc):
    return pl.pallas_call(
        _enc_body,
        grid=(1,),
        in_specs=[
            pl.BlockSpec((1, 256), lambda i: (0, 0)),
            pl.BlockSpec((256, 256), lambda i: (0, 0)),
            pl.BlockSpec((1, 256), lambda i: (0, 0)),
        ],
        out_specs=pl.BlockSpec((1, 256), lambda i: (0, 0)),
        out_shape=jax.ShapeDtypeStruct((1, 256), jnp.float32),
    )(mx, wc, bc)


def _g0_body(v_ref, enc_ref, w0_ref, b0_ref, w1_ref, b1_ref, v0_ref, *v1_refs):
    v = v_ref[...]
    x = jnp.concatenate(
        [v, jnp.broadcast_to(enc_ref[...], (v.shape[0], 256))], axis=1)
    v0_ref[...] = _mm(x, w0_ref[...], b0_ref[...])
    v1 = _mm(x, w1_ref[...], b1_ref[...])
    for q in range(4):
        v1_refs[q][...] = v1[:, q * _CW:(q + 1) * _CW]


def _g0(verts, enc, w0, b0, w1, b1):
    return pl.pallas_call(
        _g0_body,
        grid=(_GRID,),
        in_specs=[
            pl.BlockSpec((_BN, 3), lambda i: (i, 0)),
            pl.BlockSpec((1, 256), lambda i: (0, 0)),
            pl.BlockSpec((259, 256), lambda i: (0, 0)),
            pl.BlockSpec((1, 256), lambda i: (0, 0)),
            pl.BlockSpec((259, 256), lambda i: (0, 0)),
            pl.BlockSpec((1, 256), lambda i: (0, 0)),
        ],
        out_specs=[pl.BlockSpec((_BN, 256), lambda i: (i, 0))]
        + [pl.BlockSpec((_BN, _CW), lambda i: (i, 0))] * 4,
        out_shape=[jax.ShapeDtypeStruct((_N, 256), jnp.float32)]
        + [jax.ShapeDtypeStruct((_N, _CW), jnp.float32)] * 4,
    )(verts, enc, w0, b0, w1, b1)


def _agg_from_parts(p):
    return jnp.concatenate([p[0, q] + p[1, q] for q in range(4)], axis=1)


def _gt_body(v0p_ref, p_ref, w0_ref, b0_ref, w1_ref, b1_ref,
             v0_ref, *v1_refs):
    feat = jnp.maximum(v0p_ref[...] + _agg_from_parts(p_ref[...]), 0.0)
    v0_ref[...] = _mm(feat, w0_ref[...], b0_ref[...])
    v1 = _mm(feat, w1_ref[...], b1_ref[...])
    for q in range(4):
        v1_refs[q][...] = v1[:, q * _CW:(q + 1) * _CW]


def _gt(v0p, parts, w0, b0, w1, b1):
    return pl.pallas_call(
        _gt_body,
        grid=(_GRID,),
        in_specs=[
            pl.BlockSpec((_BN, 256), lambda i: (i, 0)),
            pl.BlockSpec((2, 4, _BN, _CW), lambda i: (0, 0, i, 0)),
            pl.BlockSpec((256, 256), lambda i: (0, 0)),
            pl.BlockSpec((1, 256), lambda i: (0, 0)),
            pl.BlockSpec((256, 256), lambda i: (0, 0)),
            pl.BlockSpec((1, 256), lambda i: (0, 0)),
        ],
        out_specs=[pl.BlockSpec((_BN, 256), lambda i: (i, 0))]
        + [pl.BlockSpec((_BN, _CW), lambda i: (i, 0))] * 4,
        out_shape=[jax.ShapeDtypeStruct((_N, 256), jnp.float32)]
        + [jax.ShapeDtypeStruct((_N, _CW), jnp.float32)] * 4,
    )(v0p, parts, w0, b0, w1, b1)


def _head_body(v0p_ref, p_ref, w1_ref, b1_ref, w2_ref, b2_ref, w3_ref, b3_ref,
               w4_ref, b4_ref, w5_ref, b5_ref, out_ref):
    x = jnp.maximum(v0p_ref[...] + _agg_from_parts(p_ref[...]), 0.0)
    x = jnp.maximum(_mm(x, w1_ref[...], b1_ref[...]), 0.0)
    x = jnp.maximum(_mm(x, w2_ref[...], b2_ref[...]), 0.0)
    x = jnp.maximum(_mm(x, w3_ref[...], b3_ref[...]), 0.0)
    x = jnp.maximum(_mm(x, w4_ref[...], b4_ref[...]), 0.0)
    out_ref[...] = _mm(x, w5_ref[...], b5_ref[...])


def _head(v0p, parts, ws):
    (w1, b1), (w2, b2), (w3, b3), (w4, b4), (w5, b5) = ws
    return pl.pallas_call(
        _head_body,
        grid=(_GRID,),
        in_specs=[
            pl.BlockSpec((_BN, 256), lambda i: (i, 0)),
            pl.BlockSpec((2, 4, _BN, _CW), lambda i: (0, 0, i, 0)),
            pl.BlockSpec((256, 512), lambda i: (0, 0)),
            pl.BlockSpec((1, 512), lambda i: (0, 0)),
            pl.BlockSpec((512, 512), lambda i: (0, 0)),
            pl.BlockSpec((1, 512), lambda i: (0, 0)),
            pl.BlockSpec((512, 512), lambda i: (0, 0)),
            pl.BlockSpec((1, 512), lambda i: (0, 0)),
            pl.BlockSpec((512, 128), lambda i: (0, 0)),
            pl.BlockSpec((1, 128), lambda i: (0, 0)),
            pl.BlockSpec((128, 3), lambda i: (0, 0)),
            pl.BlockSpec((1, 3), lambda i: (0, 0)),
        ],
        out_specs=pl.BlockSpec((_BN, 3), lambda i: (i, 0)),
        out_shape=jax.ShapeDtypeStruct((_N, 3), jnp.float32),
    )(v0p, parts, w1, b1, w2, b2, w3, b3, w4, b4, w5, b5)


def _edge_plan(edges):
    i0 = edges[:, 0]
    i1 = edges[:, 1]
    dsts = jnp.concatenate([i0, i1])
    srcs = jnp.concatenate([i1, i0])
    pad = _NW * _EW - dsts.shape[0]
    dsts = jnp.concatenate([dsts, jnp.full((pad,), _DUMP, jnp.int32)])
    srcs = jnp.concatenate([srcs, jnp.zeros((pad,), jnp.int32)])
    return (srcs.reshape(_NW, _NBLK, _KB), dsts.reshape(_NW, _NBLK, _KB))


def _row(b):
    return b.reshape(1, -1)


def kernel(mesh_verts, edges, params):
    verts = mesh_verts.reshape(_N, 3)
    srcs, dsts = _edge_plan(edges)

    pn = params["pointnet"]
    blk = pn["blocks"]
    net, mx = _pn1(
        verts,
        pn["fc_pos"]["w"], _row(pn["fc_pos"]["b"]),
        blk[0]["fc_0"]["w"], _row(blk[0]["fc_0"]["b"]),
        blk[0]["fc_1"]["w"], _row(blk[0]["fc_1"]["b"]),
        blk[0]["shortcut"]["w"],
    )
    for k in range(1, 5):
        net, mx = _pnk(
            net, mx,
            blk[k]["fc_0"]["w"], _row(blk[k]["fc_0"]["b"]),
            blk[k]["fc_1"]["w"], _row(blk[k]["fc_1"]["b"]),
            blk[k]["shortcut"]["w"],
        )

    g = params["gconvs"]
    enc = _enc(mx, pn["fc_c"]["w"], _row(pn["fc_c"]["b"]))
    v0, *v1q = _g0(verts, enc,
                   g[0]["w0"]["w"], _row(g[0]["w0"]["b"]),
                   g[0]["w1"]["w"], _row(g[0]["w1"]["b"]))

    for t in range(1, 11):
        parts = _sc_agg(*v1q, srcs, dsts)
        v0, *v1q = _gt(
            v0, parts,
            g[t]["w0"]["w"], _row(g[t]["w0"]["b"]),
            g[t]["w1"]["w"], _row(g[t]["w1"]["b"]),
        )

    parts = _sc_agg(*v1q, srcs, dsts)
    ws = [(p["w"], _row(p["b"])) for p in params["vert_offset"]]
    return _head(v0, parts, ws)
